# Initial kernel scaffold; baseline (speedup 1.0000x reference)
import jax
import jax.numpy as jnp
from jax.experimental import pallas as pl


def kernel(x, edge_index, edge_attr, W0, b0, W1, b1, ln0_g, ln0_b, ln1_g, ln1_b, cW1, cb1, cW2, cb2):
    raise NotImplementedError("write your pallas kernel here")


# speedup vs baseline: 2.9362x; 2.9362x over previous
import functools

import jax
import jax.numpy as jnp
from jax import lax
from jax.experimental import pallas as pl
from jax.experimental.pallas import tpu as pltpu
from jax.experimental.pallas import tpu_sc as plsc

N = 10000
E = 320000
D = 128
H1 = 128
H2 = 64
OUT = 2

NC = 2
NS = 16
NW = NC * NS
EPW = E // NW
NPAD = 10112
RPS = NPAD // NS
DC = D + 16

IB = 80
IR = EPW // IB
RB = 5
CH = RB * IB
NCH = EPW // CH

_mesh = plsc.VectorSubcoreMesh(core_axis_name="c", subcore_axis_name="s")
_sc_params = pltpu.CompilerParams(use_tc_tiling_on_sc=False)


@functools.partial(
    pl.kernel,
    out_type=jax.ShapeDtypeStruct((NC, NPAD, DC), jnp.float32),
    mesh=_mesh,
    scratch_types=[
        pltpu.VMEM((2, IB, DC), jnp.float32),
        pltpu.VMEM((IR, IB), jnp.int32),
        pltpu.VMEM_SHARED((NPAD, DC), jnp.float32),
        pltpu.SemaphoreType.DMA,
        pltpu.SemaphoreType.DMA,
        pltpu.SemaphoreType.DMA,
        pltpu.SemaphoreType.DMA,
    ],
    compiler_params=_sc_params,
)
def _segsum_sc(attr_hbm, dst_hbm, zpad_hbm, acc_out,
               rows_v, idx_v, acc_sh, ssem0, ssem1, asem0, asem1):
    cid = lax.axis_index("c")
    sid = lax.axis_index("s")
    wid = sid * NC + cid
    ssem = (ssem0, ssem1)
    asem = (asem0, asem1)

    pltpu.sync_copy(zpad_hbm.at[pl.ds(sid * RPS, RPS)],
                    acc_sh.at[pl.ds(sid * RPS, RPS)])

    pltpu.sync_copy(dst_hbm.at[wid], idx_v)

    for b in range(2):
        @pl.loop(0, IB)
        def _(r):
            rows_v[b, r, pl.ds(D, 16)] = jnp.full((16,), 1.0, jnp.float32)

    plsc.subcore_barrier()

    def stage(k, g):
        base = wid * EPW + g * CH + k * IB
        return pltpu.async_copy(
            attr_hbm.at[pl.ds(base, IB)],
            rows_v.at[k % 2, slice(None), pl.ds(0, D)], ssem[k % 2])

    def add(k, g):
        return pltpu.async_copy(
            rows_v.at[k % 2], acc_sh.at[idx_v.at[g * RB + k]],
            asem[k % 2], add=True)

    @pl.loop(0, NCH)
    def _(g):
        st = {0: stage(0, g), 1: stage(1, g)}
        ad = {}
        for k in range(RB):
            if k >= 2:
                ad[k - 2].wait()
                st[k] = stage(k, g)
            st[k].wait()
            ad[k] = add(k, g)
        ad[RB - 2].wait()
        ad[RB - 1].wait()

    plsc.subcore_barrier()

    pltpu.sync_copy(acc_sh.at[pl.ds(sid * RPS, RPS)],
                    acc_out.at[cid, pl.ds(sid * RPS, RPS)])


@functools.partial(
    pl.kernel,
    out_type=(
        jax.ShapeDtypeStruct((E, H2), jnp.float32),
        jax.ShapeDtypeStruct((E, H2), jnp.float32),
    ),
    mesh=_mesh,
    scratch_types=[
        pltpu.VMEM((IR, IB), jnp.int32),
        pltpu.VMEM((IR, IB), jnp.int32),
        pltpu.VMEM((CH, H2), jnp.float32),
        pltpu.VMEM((CH, H2), jnp.float32),
        pltpu.SemaphoreType.DMA,
        pltpu.SemaphoreType.DMA,
    ],
    compiler_params=_sc_params,
)
def _gather_sc(a_hbm, b_hbm, src_hbm, dst_hbm, e1_out, e2_out,
               si_v, di_v, bufa_v, bufb_v, sema, semb):
    cid = lax.axis_index("c")
    sid = lax.axis_index("s")
    wid = sid * NC + cid

    pltpu.sync_copy(src_hbm.at[wid], si_v)
    pltpu.sync_copy(dst_hbm.at[wid], di_v)

    @pl.loop(0, NCH)
    def _(i):
        ebase = wid * EPW + i * CH
        copies = []
        for j in range(RB):
            copies.append(pltpu.async_copy(
                a_hbm.at[si_v.at[i * RB + j]],
                bufa_v.at[pl.ds(j * IB, IB)], sema))
            copies.append(pltpu.async_copy(
                b_hbm.at[di_v.at[i * RB + j]],
                bufb_v.at[pl.ds(j * IB, IB)], semb))
        for c in copies:
            c.wait()
        pltpu.sync_copy(bufa_v, e1_out.at[pl.ds(ebase, CH)])
        pltpu.sync_copy(bufb_v, e2_out.at[pl.ds(ebase, CH)])


BN = 1000


def _mlp_body(acc_ref, x_ref, w0x_ref, w0a_ref, b0_ref,
              g0_ref, be0_ref, w1h_ref, w1a_ref, b1_ref, g1_ref, be1_ref,
              c1a_ref, cb1_ref, c1b_ref, a_ref, bt_ref):
    f = acc_ref[0] + acc_ref[1]
    s = f[:, :D]
    cnt = f[:, D:D + 1]
    agg = s / jnp.maximum(cnt, 1.0)

    h = (jnp.dot(x_ref[...], w0x_ref[...], preferred_element_type=jnp.float32)
         + jnp.dot(agg, w0a_ref[...], preferred_element_type=jnp.float32)
         + b0_ref[...])
    h = jnp.maximum(h, 0.0)
    m = jnp.mean(h, axis=-1, keepdims=True)
    v = jnp.mean((h - m) * (h - m), axis=-1, keepdims=True)
    h = (h - m) * lax.rsqrt(v + 1e-5) * g0_ref[...] + be0_ref[...]

    h = (jnp.dot(h, w1h_ref[...], preferred_element_type=jnp.float32)
         + jnp.dot(agg, w1a_ref[...], preferred_element_type=jnp.float32)
         + b1_ref[...])
    h = jnp.maximum(h, 0.0)
    m = jnp.mean(h, axis=-1, keepdims=True)
    v = jnp.mean((h - m) * (h - m), axis=-1, keepdims=True)
    h = (h - m) * lax.rsqrt(v + 1e-5) * g1_ref[...] + be1_ref[...]

    a_ref[...] = (jnp.dot(h, c1a_ref[...], preferred_element_type=jnp.float32)
                  + cb1_ref[...])
    bt_ref[...] = jnp.dot(h, c1b_ref[...], preferred_element_type=jnp.float32)


def _node_mlp(acc, x, w0x, w0a, b0, g0, be0, w1h, w1a, b1, g1, be1,
              c1a, cb1, c1b):
    full = lambda shape: pl.BlockSpec(shape, lambda i: (0,) * len(shape))
    return pl.pallas_call(
        _mlp_body,
        grid=(N // BN,),
        in_specs=[
            pl.BlockSpec((NC, BN, DC), lambda i: (0, i, 0)),
            pl.BlockSpec((BN, D), lambda i: (i, 0)),
            full((D, H1)), full((D, H1)), full((1, H1)),
            full((1, H1)), full((1, H1)),
            full((H1, H2)), full((D, H2)), full((1, H2)),
            full((1, H2)), full((1, H2)),
            full((H2, H2)), full((1, H2)), full((H2, H2)),
        ],
        out_specs=[
            pl.BlockSpec((BN, H2), lambda i: (i, 0)),
            pl.BlockSpec((BN, H2), lambda i: (i, 0)),
        ],
        out_shape=[
            jax.ShapeDtypeStruct((N, H2), jnp.float32),
            jax.ShapeDtypeStruct((N, H2), jnp.float32),
        ],
        compiler_params=pltpu.CompilerParams(
            dimension_semantics=("parallel",)),
    )(acc, x, w0x, w0a, b0, g0, be0, w1h, w1a, b1, g1, be1,
      c1a, cb1, c1b)


BE = 2000


def _cls_body(e1_ref, e2_ref, w2_ref, cb2_ref, out_ref):
    hid = jnp.maximum(e1_ref[...] + e2_ref[...], 0.0)
    o0 = jnp.sum(hid * w2_ref[0:1, :], axis=-1, keepdims=True)
    o1 = jnp.sum(hid * w2_ref[1:2, :], axis=-1, keepdims=True)
    out_ref[...] = jnp.concatenate([o0, o1], axis=-1) + cb2_ref[...]


def _edge_cls(e1, e2, cw2, cb2):
    return pl.pallas_call(
        _cls_body,
        grid=(E // BE,),
        in_specs=[
            pl.BlockSpec((BE, H2), lambda i: (i, 0)),
            pl.BlockSpec((BE, H2), lambda i: (i, 0)),
            pl.BlockSpec((OUT, H2), lambda i: (0, 0)),
            pl.BlockSpec((1, OUT), lambda i: (0, 0)),
        ],
        out_specs=pl.BlockSpec((BE, OUT), lambda i: (i, 0)),
        out_shape=jax.ShapeDtypeStruct((E, OUT), jnp.float32),
        compiler_params=pltpu.CompilerParams(
            dimension_semantics=("parallel",)),
    )(e1, e2, cw2, cb2)


def kernel(x, edge_index, edge_attr, W0, b0, W1, b1, ln0_g, ln0_b,
           ln1_g, ln1_b, cW1, cb1, cW2, cb2):
    src = edge_index[0].astype(jnp.int32).reshape(NW, IR, IB)
    dst = edge_index[1].astype(jnp.int32).reshape(NW, IR, IB)

    zeros_pad = jnp.zeros((NPAD, DC), jnp.float32)
    acc = _segsum_sc(edge_attr, dst, zeros_pad)

    a_tab, b_tab = _node_mlp(
        acc, x,
        W0[:, :D].T, W0[:, D:].T, b0.reshape(1, H1),
        ln0_g.reshape(1, H1), ln0_b.reshape(1, H1),
        W1[:, :H1].T, W1[:, H1:].T, b1.reshape(1, H2),
        ln1_g.reshape(1, H2), ln1_b.reshape(1, H2),
        cW1[:, :H2].T, cb1.reshape(1, H2), cW1[:, H2:].T,
    )

    e1, e2 = _gather_sc(a_tab, b_tab, src, dst)

    return _edge_cls(e1, e2, cW2, cb2.reshape(1, OUT))


# speedup vs baseline: 4.1684x; 1.4197x over previous
import functools

import jax
import jax.numpy as jnp
from jax import lax
from jax.experimental import pallas as pl
from jax.experimental.pallas import tpu as pltpu
from jax.experimental.pallas import tpu_sc as plsc

N = 10000
E = 320000
D = 128
H1 = 128
H2 = 64
OUT = 2

NC = 2
NS = 16
NW = NC * NS
EPW = E // NW
NPAD = 10112
RPS = NPAD // NS
DC = D + 16

IB = 80
IR = EPW // IB
RB = 5
CH = RB * IB
NCH = EPW // CH

_mesh = plsc.VectorSubcoreMesh(core_axis_name="c", subcore_axis_name="s")
_sc_params = pltpu.CompilerParams(use_tc_tiling_on_sc=False)


@functools.partial(
    pl.kernel,
    out_type=(
        jax.ShapeDtypeStruct((NC, NPAD, D), jnp.float32),
        jax.ShapeDtypeStruct((NC, NPAD, 16), jnp.float32),
    ),
    mesh=_mesh,
    scratch_types=[
        pltpu.VMEM((2, IB, DC), jnp.float32),
        pltpu.VMEM((IR, IB), jnp.int32),
        pltpu.VMEM_SHARED((NPAD, DC), jnp.float32),
        pltpu.SemaphoreType.DMA,
        pltpu.SemaphoreType.DMA,
        pltpu.SemaphoreType.DMA,
        pltpu.SemaphoreType.DMA,
    ],
    compiler_params=_sc_params,
)
def _segsum_sc(attr_hbm, dst_hbm, zpad_hbm, sums_out, cnts_out,
               rows_v, idx_v, acc_sh, ssem0, ssem1, asem0, asem1):
    cid = lax.axis_index("c")
    sid = lax.axis_index("s")
    wid = sid * NC + cid
    ssem = (ssem0, ssem1)
    asem = (asem0, asem1)

    pltpu.sync_copy(zpad_hbm.at[pl.ds(sid * RPS, RPS)],
                    acc_sh.at[pl.ds(sid * RPS, RPS)])

    pltpu.sync_copy(dst_hbm.at[wid], idx_v)

    for b in range(2):
        @pl.loop(0, IB)
        def _(r):
            rows_v[b, r, pl.ds(D, 16)] = jnp.full((16,), 1.0, jnp.float32)

    plsc.subcore_barrier()

    def stage(k, g):
        base = wid * EPW + g * CH + k * IB
        return pltpu.async_copy(
            attr_hbm.at[pl.ds(base, IB)],
            rows_v.at[k % 2, slice(None), pl.ds(0, D)], ssem[k % 2])

    def add(k, g):
        return pltpu.async_copy(
            rows_v.at[k % 2], acc_sh.at[idx_v.at[g * RB + k]],
            asem[k % 2], add=True)

    @pl.loop(0, NCH)
    def _(g):
        st = {0: stage(0, g), 1: stage(1, g)}
        ad = {}
        for k in range(RB):
            if k >= 2:
                ad[k - 2].wait()
                st[k] = stage(k, g)
            st[k].wait()
            ad[k] = add(k, g)
        ad[RB - 2].wait()
        ad[RB - 1].wait()

    plsc.subcore_barrier()

    pltpu.sync_copy(acc_sh.at[pl.ds(sid * RPS, RPS), pl.ds(0, D)],
                    sums_out.at[cid, pl.ds(sid * RPS, RPS)])
    pltpu.sync_copy(acc_sh.at[pl.ds(sid * RPS, RPS), pl.ds(D, 16)],
                    cnts_out.at[cid, pl.ds(sid * RPS, RPS)])


@functools.partial(
    pl.kernel,
    out_type=jax.ShapeDtypeStruct((E, 2 * H2), jnp.float32),
    mesh=_mesh,
    scratch_types=[
        pltpu.VMEM((IR, IB), jnp.int32),
        pltpu.VMEM((IR, IB), jnp.int32),
        pltpu.VMEM((CH, H2), jnp.float32),
        pltpu.VMEM((CH, H2), jnp.float32),
        pltpu.SemaphoreType.DMA,
        pltpu.SemaphoreType.DMA,
    ],
    compiler_params=_sc_params,
)
def _gather_sc(a_hbm, b_hbm, src_hbm, dst_hbm, e_out,
               si_v, di_v, bufa_v, bufb_v, sema, semb):
    cid = lax.axis_index("c")
    sid = lax.axis_index("s")
    wid = sid * NC + cid

    pltpu.sync_copy(src_hbm.at[wid], si_v)
    pltpu.sync_copy(dst_hbm.at[wid], di_v)

    @pl.loop(0, NCH)
    def _(i):
        ebase = wid * EPW + i * CH
        copies = []
        for j in range(RB):
            copies.append(pltpu.async_copy(
                a_hbm.at[si_v.at[i * RB + j]],
                bufa_v.at[pl.ds(j * IB, IB)], sema))
            copies.append(pltpu.async_copy(
                b_hbm.at[di_v.at[i * RB + j]],
                bufb_v.at[pl.ds(j * IB, IB)], semb))
        for c in copies:
            c.wait()
        pltpu.sync_copy(bufa_v, e_out.at[pl.ds(ebase, CH), pl.ds(0, H2)])
        pltpu.sync_copy(bufb_v, e_out.at[pl.ds(ebase, CH), pl.ds(H2, H2)])


BN = 1000


def _mlp_body(sums_ref, cnts_ref, x_ref, w0x_ref, w0a_ref, b0_ref,
              g0_ref, be0_ref, w1h_ref, w1a_ref, b1_ref, g1_ref, be1_ref,
              c1a_ref, cb1_ref, c1b_ref, a_ref, bt_ref):
    s = sums_ref[0] + sums_ref[1]
    cnt = cnts_ref[0, :, 0:1] + cnts_ref[1, :, 0:1]
    agg = s / jnp.maximum(cnt, 1.0)

    h = (jnp.dot(x_ref[...], w0x_ref[...], preferred_element_type=jnp.float32)
         + jnp.dot(agg, w0a_ref[...], preferred_element_type=jnp.float32)
         + b0_ref[...])
    h = jnp.maximum(h, 0.0)
    m = jnp.mean(h, axis=-1, keepdims=True)
    v = jnp.mean((h - m) * (h - m), axis=-1, keepdims=True)
    h = (h - m) * lax.rsqrt(v + 1e-5) * g0_ref[...] + be0_ref[...]

    h = (jnp.dot(h, w1h_ref[...], preferred_element_type=jnp.float32)
         + jnp.dot(agg, w1a_ref[...], preferred_element_type=jnp.float32)
         + b1_ref[...])
    h = jnp.maximum(h, 0.0)
    m = jnp.mean(h, axis=-1, keepdims=True)
    v = jnp.mean((h - m) * (h - m), axis=-1, keepdims=True)
    h = (h - m) * lax.rsqrt(v + 1e-5) * g1_ref[...] + be1_ref[...]

    a_ref[...] = (jnp.dot(h, c1a_ref[...], preferred_element_type=jnp.float32)
                  + cb1_ref[...])
    bt_ref[...] = jnp.dot(h, c1b_ref[...], preferred_element_type=jnp.float32)


def _node_mlp(sums, cnts, x, w0x, w0a, b0, g0, be0, w1h, w1a, b1, g1, be1,
              c1a, cb1, c1b):
    full = lambda shape: pl.BlockSpec(shape, lambda i: (0,) * len(shape))
    return pl.pallas_call(
        _mlp_body,
        grid=(N // BN,),
        in_specs=[
            pl.BlockSpec((NC, BN, D), lambda i: (0, i, 0)),
            pl.BlockSpec((NC, BN, 16), lambda i: (0, i, 0)),
            pl.BlockSpec((BN, D), lambda i: (i, 0)),
            full((D, H1)), full((D, H1)), full((1, H1)),
            full((1, H1)), full((1, H1)),
            full((H1, H2)), full((D, H2)), full((1, H2)),
            full((1, H2)), full((1, H2)),
            full((H2, H2)), full((1, H2)), full((H2, H2)),
        ],
        out_specs=[
            pl.BlockSpec((BN, H2), lambda i: (i, 0)),
            pl.BlockSpec((BN, H2), lambda i: (i, 0)),
        ],
        out_shape=[
            jax.ShapeDtypeStruct((N, H2), jnp.float32),
            jax.ShapeDtypeStruct((N, H2), jnp.float32),
        ],
        compiler_params=pltpu.CompilerParams(
            dimension_semantics=("parallel",)),
    )(sums, cnts, x, w0x, w0a, b0, g0, be0, w1h, w1a, b1, g1, be1,
      c1a, cb1, c1b)


BE = 2000


def _cls_body(e_ref, w2_ref, cb2_ref, out_ref):
    e = e_ref[...]
    hid = jnp.maximum(e[:, :H2] + e[:, H2:], 0.0)
    o0 = jnp.sum(hid * w2_ref[0:1, :], axis=-1, keepdims=True)
    o1 = jnp.sum(hid * w2_ref[1:2, :], axis=-1, keepdims=True)
    out_ref[...] = jnp.concatenate([o0, o1], axis=-1) + cb2_ref[...]


def _edge_cls(e12, cw2, cb2):
    return pl.pallas_call(
        _cls_body,
        grid=(E // BE,),
        in_specs=[
            pl.BlockSpec((BE, 2 * H2), lambda i: (i, 0)),
            pl.BlockSpec((OUT, H2), lambda i: (0, 0)),
            pl.BlockSpec((1, OUT), lambda i: (0, 0)),
        ],
        out_specs=pl.BlockSpec((BE, OUT), lambda i: (i, 0)),
        out_shape=jax.ShapeDtypeStruct((E, OUT), jnp.float32),
        compiler_params=pltpu.CompilerParams(
            dimension_semantics=("parallel",)),
    )(e12, cw2, cb2)


def kernel(x, edge_index, edge_attr, W0, b0, W1, b1, ln0_g, ln0_b,
           ln1_g, ln1_b, cW1, cb1, cW2, cb2):
    src = edge_index[0].astype(jnp.int32).reshape(NW, IR, IB)
    dst = edge_index[1].astype(jnp.int32).reshape(NW, IR, IB)

    zeros_pad = jnp.zeros((NPAD, DC), jnp.float32)
    sums, cnts = _segsum_sc(edge_attr, dst, zeros_pad)

    a_tab, b_tab = _node_mlp(
        sums, cnts, x,
        W0[:, :D].T, W0[:, D:].T, b0.reshape(1, H1),
        ln0_g.reshape(1, H1), ln0_b.reshape(1, H1),
        W1[:, :H1].T, W1[:, H1:].T, b1.reshape(1, H2),
        ln1_g.reshape(1, H2), ln1_b.reshape(1, H2),
        cW1[:, :H2].T, cb1.reshape(1, H2), cW1[:, H2:].T,
    )

    e12 = _gather_sc(a_tab, b_tab, src, dst)

    return _edge_cls(e12, cW2, cb2.reshape(1, OUT))


# speedup vs baseline: 4.7635x; 1.1428x over previous
import functools

import jax
import jax.numpy as jnp
from jax import lax
from jax.experimental import pallas as pl
from jax.experimental.pallas import tpu as pltpu
from jax.experimental.pallas import tpu_sc as plsc

N = 10000
E = 320000
D = 128
H1 = 128
H2 = 64
OUT = 2

NC = 2
NS = 16
NW = NC * NS
EPW = E // NW
NPAD = 10112
RPS = NPAD // NS
DC = D + 16

IB = 80
IR = EPW // IB
RB = 5
CH = RB * IB
NCH = EPW // CH

_mesh = plsc.VectorSubcoreMesh(core_axis_name="c", subcore_axis_name="s")
_sc_params = pltpu.CompilerParams(use_tc_tiling_on_sc=False)


@functools.partial(
    pl.kernel,
    out_type=(
        jax.ShapeDtypeStruct((NC, NPAD, D), jnp.float32),
        jax.ShapeDtypeStruct((NC, NPAD, 16), jnp.float32),
    ),
    mesh=_mesh,
    scratch_types=[
        pltpu.VMEM((2, IB, DC), jnp.float32),
        pltpu.VMEM((IR, IB), jnp.int32),
        pltpu.VMEM_SHARED((NPAD, DC), jnp.float32),
        pltpu.SemaphoreType.DMA,
        pltpu.SemaphoreType.DMA,
        pltpu.SemaphoreType.DMA,
        pltpu.SemaphoreType.DMA,
    ],
    compiler_params=_sc_params,
)
def _segsum_sc(attr_hbm, dst_hbm, zpad_hbm, sums_out, cnts_out,
               rows_v, idx_v, acc_sh, ssem0, ssem1, asem0, asem1):
    cid = lax.axis_index("c")
    sid = lax.axis_index("s")
    wid = sid * NC + cid
    ssem = (ssem0, ssem1)
    asem = (asem0, asem1)

    pltpu.sync_copy(zpad_hbm.at[pl.ds(sid * RPS, RPS)],
                    acc_sh.at[pl.ds(sid * RPS, RPS)])

    pltpu.sync_copy(dst_hbm.at[wid], idx_v)

    for b in range(2):
        @pl.loop(0, IB)
        def _(r):
            rows_v[b, r, pl.ds(D, 16)] = jnp.full((16,), 1.0, jnp.float32)

    plsc.subcore_barrier()

    def stage(k, g):
        base = wid * EPW + g * CH + k * IB
        return pltpu.async_copy(
            attr_hbm.at[pl.ds(base, IB)],
            rows_v.at[k % 2, slice(None), pl.ds(0, D)], ssem[k % 2])

    def add(k, g):
        return pltpu.async_copy(
            rows_v.at[k % 2], acc_sh.at[idx_v.at[g * RB + k]],
            asem[k % 2], add=True)

    @pl.loop(0, NCH)
    def _(g):
        st = {0: stage(0, g), 1: stage(1, g)}
        ad = {}
        for k in range(RB):
            if k >= 2:
                ad[k - 2].wait()
                st[k] = stage(k, g)
            st[k].wait()
            ad[k] = add(k, g)
        ad[RB - 2].wait()
        ad[RB - 1].wait()

    plsc.subcore_barrier()

    pltpu.sync_copy(acc_sh.at[pl.ds(sid * RPS, RPS), pl.ds(0, D)],
                    sums_out.at[cid, pl.ds(sid * RPS, RPS)])
    pltpu.sync_copy(acc_sh.at[pl.ds(sid * RPS, RPS), pl.ds(D, 16)],
                    cnts_out.at[cid, pl.ds(sid * RPS, RPS)])


@functools.partial(
    pl.kernel,
    out_type=jax.ShapeDtypeStruct((E, 2 * H2), jnp.float32),
    mesh=_mesh,
    scratch_types=[
        pltpu.VMEM((IR, IB), jnp.int32),
        pltpu.VMEM((IR, IB), jnp.int32),
        pltpu.VMEM((CH, H2), jnp.float32),
        pltpu.VMEM((CH, H2), jnp.float32),
        pltpu.SemaphoreType.DMA,
        pltpu.SemaphoreType.DMA,
    ],
    compiler_params=_sc_params,
)
def _gather_sc(a_hbm, b_hbm, src_hbm, dst_hbm, e_out,
               si_v, di_v, bufa_v, bufb_v, sema, semb):
    cid = lax.axis_index("c")
    sid = lax.axis_index("s")
    wid = sid * NC + cid

    pltpu.sync_copy(src_hbm.at[wid], si_v)
    pltpu.sync_copy(dst_hbm.at[wid], di_v)

    @pl.loop(0, NCH)
    def _(i):
        ebase = wid * EPW + i * CH
        copies = []
        for j in range(RB):
            copies.append(pltpu.async_copy(
                a_hbm.at[si_v.at[i * RB + j]],
                bufa_v.at[pl.ds(j * IB, IB)], sema))
            copies.append(pltpu.async_copy(
                b_hbm.at[di_v.at[i * RB + j]],
                bufb_v.at[pl.ds(j * IB, IB)], semb))
        for c in copies:
            c.wait()
        pltpu.sync_copy(bufa_v, e_out.at[pl.ds(ebase, CH), pl.ds(0, H2)])
        pltpu.sync_copy(bufb_v, e_out.at[pl.ds(ebase, CH), pl.ds(H2, H2)])


BN = 1000


def _mlp_body(sums_ref, cnts_ref, x_ref, w0x_ref, w0a_ref, b0_ref,
              g0_ref, be0_ref, w1h_ref, w1a_ref, b1_ref, g1_ref, be1_ref,
              c1a_ref, cb1_ref, c1b_ref, a_ref, bt_ref):
    s = sums_ref[0] + sums_ref[1]
    cnt = cnts_ref[0, :, 0:1] + cnts_ref[1, :, 0:1]
    agg = s / jnp.maximum(cnt, 1.0)

    h = (jnp.dot(x_ref[...], w0x_ref[...], preferred_element_type=jnp.float32)
         + jnp.dot(agg, w0a_ref[...], preferred_element_type=jnp.float32)
         + b0_ref[...])
    h = jnp.maximum(h, 0.0)
    m = jnp.mean(h, axis=-1, keepdims=True)
    v = jnp.mean((h - m) * (h - m), axis=-1, keepdims=True)
    h = (h - m) * lax.rsqrt(v + 1e-5) * g0_ref[...] + be0_ref[...]

    h = (jnp.dot(h, w1h_ref[...], preferred_element_type=jnp.float32)
         + jnp.dot(agg, w1a_ref[...], preferred_element_type=jnp.float32)
         + b1_ref[...])
    h = jnp.maximum(h, 0.0)
    m = jnp.mean(h, axis=-1, keepdims=True)
    v = jnp.mean((h - m) * (h - m), axis=-1, keepdims=True)
    h = (h - m) * lax.rsqrt(v + 1e-5) * g1_ref[...] + be1_ref[...]

    a_ref[...] = (jnp.dot(h, c1a_ref[...], preferred_element_type=jnp.float32)
                  + cb1_ref[...])
    bt_ref[...] = jnp.dot(h, c1b_ref[...], preferred_element_type=jnp.float32)


def _node_mlp(sums, cnts, x, w0x, w0a, b0, g0, be0, w1h, w1a, b1, g1, be1,
              c1a, cb1, c1b):
    full = lambda shape: pl.BlockSpec(shape, lambda i: (0,) * len(shape))
    return pl.pallas_call(
        _mlp_body,
        grid=(N // BN,),
        in_specs=[
            pl.BlockSpec((NC, BN, D), lambda i: (0, i, 0)),
            pl.BlockSpec((NC, BN, 16), lambda i: (0, i, 0)),
            pl.BlockSpec((BN, D), lambda i: (i, 0)),
            full((D, H1)), full((D, H1)), full((1, H1)),
            full((1, H1)), full((1, H1)),
            full((H1, H2)), full((D, H2)), full((1, H2)),
            full((1, H2)), full((1, H2)),
            full((H2, H2)), full((1, H2)), full((H2, H2)),
        ],
        out_specs=[
            pl.BlockSpec((BN, H2), lambda i: (i, 0)),
            pl.BlockSpec((BN, H2), lambda i: (i, 0)),
        ],
        out_shape=[
            jax.ShapeDtypeStruct((N, H2), jnp.float32),
            jax.ShapeDtypeStruct((N, H2), jnp.float32),
        ],
        compiler_params=pltpu.CompilerParams(
            dimension_semantics=("parallel",)),
    )(sums, cnts, x, w0x, w0a, b0, g0, be0, w1h, w1a, b1, g1, be1,
      c1a, cb1, c1b)


BE = 4000


def _cls_body(e_ref, w2t_ref, cb2_ref, out_ref):
    e = e_ref[...]
    hid = jnp.maximum(e[:, :H2] + e[:, H2:], 0.0)
    out_ref[...] = (jnp.dot(hid, w2t_ref[...],
                            preferred_element_type=jnp.float32)
                    + cb2_ref[...])


def _edge_cls(e12, cw2t, cb2):
    return pl.pallas_call(
        _cls_body,
        grid=(E // BE,),
        in_specs=[
            pl.BlockSpec((BE, 2 * H2), lambda i: (i, 0)),
            pl.BlockSpec((H2, OUT), lambda i: (0, 0)),
            pl.BlockSpec((1, OUT), lambda i: (0, 0)),
        ],
        out_specs=pl.BlockSpec((BE, OUT), lambda i: (i, 0)),
        out_shape=jax.ShapeDtypeStruct((E, OUT), jnp.float32),
        compiler_params=pltpu.CompilerParams(
            dimension_semantics=("parallel",)),
    )(e12, cw2t, cb2)


def kernel(x, edge_index, edge_attr, W0, b0, W1, b1, ln0_g, ln0_b,
           ln1_g, ln1_b, cW1, cb1, cW2, cb2):
    src = edge_index[0].astype(jnp.int32).reshape(NW, IR, IB)
    dst = edge_index[1].astype(jnp.int32).reshape(NW, IR, IB)

    zeros_pad = jnp.zeros((NPAD, DC), jnp.float32)
    sums, cnts = _segsum_sc(edge_attr, dst, zeros_pad)

    a_tab, b_tab = _node_mlp(
        sums, cnts, x,
        W0[:, :D].T, W0[:, D:].T, b0.reshape(1, H1),
        ln0_g.reshape(1, H1), ln0_b.reshape(1, H1),
        W1[:, :H1].T, W1[:, H1:].T, b1.reshape(1, H2),
        ln1_g.reshape(1, H2), ln1_b.reshape(1, H2),
        cW1[:, :H2].T, cb1.reshape(1, H2), cW1[:, H2:].T,
    )

    e12 = _gather_sc(a_tab, b_tab, src, dst)

    return _edge_cls(e12, cW2.T, cb2.reshape(1, OUT))


# speedup vs baseline: 6.2012x; 1.3018x over previous
import functools

import jax
import jax.numpy as jnp
from jax import lax
from jax.experimental import pallas as pl
from jax.experimental.pallas import tpu as pltpu
from jax.experimental.pallas import tpu_sc as plsc

N = 10000
E = 320000
D = 128
H1 = 128
H2 = 64
OUT = 2

NC = 2
NS = 16
NW = NC * NS
EPW = E // NW
NPAD = 10112
RPS = NPAD // NS
DC = D + 16

IB = 80
IR = EPW // IB
RB = 5
CH = RB * IB
NCH = EPW // CH

_mesh = plsc.VectorSubcoreMesh(core_axis_name="c", subcore_axis_name="s")
_sc_params = pltpu.CompilerParams(use_tc_tiling_on_sc=False)


@functools.partial(
    pl.kernel,
    out_type=(
        jax.ShapeDtypeStruct((NC, NPAD, D), jnp.float32),
        jax.ShapeDtypeStruct((NC, NPAD, 16), jnp.float32),
    ),
    mesh=_mesh,
    scratch_types=[
        pltpu.VMEM((2, IB, DC), jnp.float32),
        pltpu.VMEM((IR, IB), jnp.int32),
        pltpu.VMEM_SHARED((NPAD, DC), jnp.float32),
        pltpu.SemaphoreType.DMA,
        pltpu.SemaphoreType.DMA,
        pltpu.SemaphoreType.DMA,
        pltpu.SemaphoreType.DMA,
    ],
    compiler_params=_sc_params,
)
def _segsum_sc(attr_hbm, dst_hbm, zpad_hbm, sums_out, cnts_out,
               rows_v, idx_v, acc_sh, ssem0, ssem1, asem0, asem1):
    cid = lax.axis_index("c")
    sid = lax.axis_index("s")
    wid = sid * NC + cid
    ssem = (ssem0, ssem1)
    asem = (asem0, asem1)

    pltpu.sync_copy(zpad_hbm.at[pl.ds(sid * RPS, RPS)],
                    acc_sh.at[pl.ds(sid * RPS, RPS)])

    pltpu.sync_copy(dst_hbm.at[wid], idx_v)

    for b in range(2):
        @pl.loop(0, IB)
        def _(r):
            rows_v[b, r, pl.ds(D, 16)] = jnp.full((16,), 1.0, jnp.float32)

    plsc.subcore_barrier()

    def stage(k, g):
        base = wid * EPW + g * CH + k * IB
        return pltpu.async_copy(
            attr_hbm.at[pl.ds(base, IB)],
            rows_v.at[k % 2, slice(None), pl.ds(0, D)], ssem[k % 2])

    def add(k, g):
        return pltpu.async_copy(
            rows_v.at[k % 2], acc_sh.at[idx_v.at[g * RB + k]],
            asem[k % 2], add=True)

    @pl.loop(0, NCH)
    def _(g):
        st = {0: stage(0, g), 1: stage(1, g)}
        ad = {}
        for k in range(RB):
            if k >= 2:
                ad[k - 2].wait()
                st[k] = stage(k, g)
            st[k].wait()
            ad[k] = add(k, g)
        ad[RB - 2].wait()
        ad[RB - 1].wait()

    plsc.subcore_barrier()

    pltpu.sync_copy(acc_sh.at[pl.ds(sid * RPS, RPS), pl.ds(0, D)],
                    sums_out.at[cid, pl.ds(sid * RPS, RPS)])
    pltpu.sync_copy(acc_sh.at[pl.ds(sid * RPS, RPS), pl.ds(D, 16)],
                    cnts_out.at[cid, pl.ds(sid * RPS, RPS)])


@functools.partial(
    pl.kernel,
    out_type=jax.ShapeDtypeStruct((E, 2 * H2), jnp.float32),
    mesh=_mesh,
    scratch_types=[
        pltpu.VMEM((IR, IB), jnp.int32),
        pltpu.VMEM((IR, IB), jnp.int32),
        pltpu.VMEM((CH, H2), jnp.float32),
        pltpu.VMEM((CH, H2), jnp.float32),
        pltpu.SemaphoreType.DMA,
        pltpu.SemaphoreType.DMA,
    ],
    compiler_params=_sc_params,
)
def _gather_sc(a_hbm, b_hbm, src_hbm, dst_hbm, e_out,
               si_v, di_v, bufa_v, bufb_v, sema, semb):
    cid = lax.axis_index("c")
    sid = lax.axis_index("s")
    wid = sid * NC + cid

    pltpu.sync_copy(src_hbm.at[wid], si_v)
    pltpu.sync_copy(dst_hbm.at[wid], di_v)

    @pl.loop(0, NCH)
    def _(i):
        ebase = wid * EPW + i * CH
        copies = []
        for j in range(RB):
            copies.append(pltpu.async_copy(
                a_hbm.at[si_v.at[i * RB + j]],
                bufa_v.at[pl.ds(j * IB, IB)], sema))
            copies.append(pltpu.async_copy(
                b_hbm.at[di_v.at[i * RB + j]],
                bufb_v.at[pl.ds(j * IB, IB)], semb))
        for c in copies:
            c.wait()
        pltpu.sync_copy(bufa_v, e_out.at[pl.ds(ebase, CH), pl.ds(0, H2)])
        pltpu.sync_copy(bufb_v, e_out.at[pl.ds(ebase, CH), pl.ds(H2, H2)])


BN = 1000


def _mlp_body(sums_ref, cnts_ref, x_ref, w0x_ref, w0a_ref, b0_ref,
              g0_ref, be0_ref, w1h_ref, w1a_ref, b1_ref, g1_ref, be1_ref,
              c1a_ref, cb1_ref, c1b_ref, a_ref, bt_ref):
    s = sums_ref[0] + sums_ref[1]
    cnt = cnts_ref[0, :, 0:1] + cnts_ref[1, :, 0:1]
    agg = s / jnp.maximum(cnt, 1.0)

    h = (jnp.dot(x_ref[...], w0x_ref[...], preferred_element_type=jnp.float32)
         + jnp.dot(agg, w0a_ref[...], preferred_element_type=jnp.float32)
         + b0_ref[...])
    h = jnp.maximum(h, 0.0)
    m = jnp.mean(h, axis=-1, keepdims=True)
    v = jnp.mean((h - m) * (h - m), axis=-1, keepdims=True)
    h = (h - m) * lax.rsqrt(v + 1e-5) * g0_ref[...] + be0_ref[...]

    h = (jnp.dot(h, w1h_ref[...], preferred_element_type=jnp.float32)
         + jnp.dot(agg, w1a_ref[...], preferred_element_type=jnp.float32)
         + b1_ref[...])
    h = jnp.maximum(h, 0.0)
    m = jnp.mean(h, axis=-1, keepdims=True)
    v = jnp.mean((h - m) * (h - m), axis=-1, keepdims=True)
    h = (h - m) * lax.rsqrt(v + 1e-5) * g1_ref[...] + be1_ref[...]

    a_ref[...] = (jnp.dot(h, c1a_ref[...], preferred_element_type=jnp.float32)
                  + cb1_ref[...])
    bt_ref[...] = jnp.dot(h, c1b_ref[...], preferred_element_type=jnp.float32)


def _node_mlp(sums, cnts, x, w0x, w0a, b0, g0, be0, w1h, w1a, b1, g1, be1,
              c1a, cb1, c1b):
    full = lambda shape: pl.BlockSpec(shape, lambda i: (0,) * len(shape))
    return pl.pallas_call(
        _mlp_body,
        grid=(N // BN,),
        in_specs=[
            pl.BlockSpec((NC, BN, D), lambda i: (0, i, 0)),
            pl.BlockSpec((NC, BN, 16), lambda i: (0, i, 0)),
            pl.BlockSpec((BN, D), lambda i: (i, 0)),
            full((D, H1)), full((D, H1)), full((1, H1)),
            full((1, H1)), full((1, H1)),
            full((H1, H2)), full((D, H2)), full((1, H2)),
            full((1, H2)), full((1, H2)),
            full((H2, H2)), full((1, H2)), full((H2, H2)),
        ],
        out_specs=[
            pl.BlockSpec((BN, H2), lambda i: (i, 0)),
            pl.BlockSpec((BN, H2), lambda i: (i, 0)),
        ],
        out_shape=[
            jax.ShapeDtypeStruct((N, H2), jnp.float32),
            jax.ShapeDtypeStruct((N, H2), jnp.float32),
        ],
        compiler_params=pltpu.CompilerParams(
            dimension_semantics=("parallel",)),
    )(sums, cnts, x, w0x, w0a, b0, g0, be0, w1h, w1a, b1, g1, be1,
      c1a, cb1, c1b)


BE = 6400


def _cls_body(e_ref, w2_ref, cb2_ref, out_ref):
    e = e_ref[...]
    hid = jnp.maximum(e[:, :H2] + e[:, H2:], 0.0)
    ot = lax.dot_general(w2_ref[...], hid, (((1,), (1,)), ((), ())),
                         preferred_element_type=jnp.float32)
    out_ref[...] = ot + cb2_ref[...]


def _edge_cls(e12, cw2, cb2):
    return pl.pallas_call(
        _cls_body,
        grid=(E // BE,),
        in_specs=[
            pl.BlockSpec((BE, 2 * H2), lambda i: (i, 0)),
            pl.BlockSpec((OUT, H2), lambda i: (0, 0)),
            pl.BlockSpec((OUT, 1), lambda i: (0, 0)),
        ],
        out_specs=pl.BlockSpec((OUT, BE), lambda i: (0, i)),
        out_shape=jax.ShapeDtypeStruct((OUT, E), jnp.float32),
        compiler_params=pltpu.CompilerParams(
            dimension_semantics=("parallel",)),
    )(e12, cw2, cb2)


def kernel(x, edge_index, edge_attr, W0, b0, W1, b1, ln0_g, ln0_b,
           ln1_g, ln1_b, cW1, cb1, cW2, cb2):
    src = edge_index[0].astype(jnp.int32).reshape(NW, IR, IB)
    dst = edge_index[1].astype(jnp.int32).reshape(NW, IR, IB)

    zeros_pad = jnp.zeros((NPAD, DC), jnp.float32)
    sums, cnts = _segsum_sc(edge_attr, dst, zeros_pad)

    a_tab, b_tab = _node_mlp(
        sums, cnts, x,
        W0[:, :D].T, W0[:, D:].T, b0.reshape(1, H1),
        ln0_g.reshape(1, H1), ln0_b.reshape(1, H1),
        W1[:, :H1].T, W1[:, H1:].T, b1.reshape(1, H2),
        ln1_g.reshape(1, H2), ln1_b.reshape(1, H2),
        cW1[:, :H2].T, cb1.reshape(1, H2), cW1[:, H2:].T,
    )

    e12 = _gather_sc(a_tab, b_tab, src, dst)

    return _edge_cls(e12, cW2, cb2.reshape(OUT, 1)).T


# speedup vs baseline: 6.2645x; 1.0102x over previous
import functools

import jax
import jax.numpy as jnp
from jax import lax
from jax.experimental import pallas as pl
from jax.experimental.pallas import tpu as pltpu
from jax.experimental.pallas import tpu_sc as plsc

N = 10000
E = 320000
D = 128
H1 = 128
H2 = 64
OUT = 2

NC = 2
NS = 16
NW = NC * NS
EPW = E // NW
NPAD = 10112
RPS = NPAD // NS
DC = D + 16

IB = 80
IR = EPW // IB
RB = 5
CH = RB * IB
NCH = EPW // CH

_mesh = plsc.VectorSubcoreMesh(core_axis_name="c", subcore_axis_name="s")
_sc_params = pltpu.CompilerParams(use_tc_tiling_on_sc=False)


@functools.partial(
    pl.kernel,
    out_type=(
        jax.ShapeDtypeStruct((NC, NPAD, D), jnp.float32),
        jax.ShapeDtypeStruct((NC, NPAD, 16), jnp.float32),
    ),
    mesh=_mesh,
    scratch_types=[
        pltpu.VMEM((2, IB, DC), jnp.float32),
        pltpu.VMEM((IR, IB), jnp.int32),
        pltpu.VMEM_SHARED((NPAD, DC), jnp.float32),
        pltpu.SemaphoreType.DMA,
        pltpu.SemaphoreType.DMA,
        pltpu.SemaphoreType.DMA,
        pltpu.SemaphoreType.DMA,
    ],
    compiler_params=_sc_params,
)
def _segsum_sc(attr_hbm, dst_hbm, zpad_hbm, sums_out, cnts_out,
               rows_v, idx_v, acc_sh, ssem0, ssem1, asem0, asem1):
    cid = lax.axis_index("c")
    sid = lax.axis_index("s")
    wid = sid * NC + cid
    ssem = (ssem0, ssem1)
    asem = (asem0, asem1)

    pltpu.sync_copy(zpad_hbm.at[pl.ds(sid * RPS, RPS)],
                    acc_sh.at[pl.ds(sid * RPS, RPS)])

    pltpu.sync_copy(dst_hbm.at[wid], idx_v)

    for b in range(2):
        @pl.loop(0, IB)
        def _(r):
            rows_v[b, r, pl.ds(D, 16)] = jnp.full((16,), 1.0, jnp.float32)

    plsc.subcore_barrier()

    def stage(k, g):
        base = wid * EPW + g * CH + k * IB
        return pltpu.async_copy(
            attr_hbm.at[pl.ds(base, IB)],
            rows_v.at[k % 2, slice(None), pl.ds(0, D)], ssem[k % 2])

    def add(k, g):
        return pltpu.async_copy(
            rows_v.at[k % 2], acc_sh.at[idx_v.at[g * RB + k]],
            asem[k % 2], add=True)

    @pl.loop(0, NCH)
    def _(g):
        st = {0: stage(0, g), 1: stage(1, g)}
        ad = {}
        for k in range(RB):
            if k >= 2:
                ad[k - 2].wait()
                st[k] = stage(k, g)
            st[k].wait()
            ad[k] = add(k, g)
        ad[RB - 2].wait()
        ad[RB - 1].wait()

    plsc.subcore_barrier()

    pltpu.sync_copy(acc_sh.at[pl.ds(sid * RPS, RPS), pl.ds(0, D)],
                    sums_out.at[cid, pl.ds(sid * RPS, RPS)])
    pltpu.sync_copy(acc_sh.at[pl.ds(sid * RPS, RPS), pl.ds(D, 16)],
                    cnts_out.at[cid, pl.ds(sid * RPS, RPS)])


SLABS = 5
SLAB_E = E // SLABS
EPWS = SLAB_E // NW
IRS = EPWS // IB
NCHS = EPWS // CH


def _make_gather(slab):
    @functools.partial(
        pl.kernel,
        out_type=jax.ShapeDtypeStruct((SLAB_E, 2 * H2), jnp.float32),
        mesh=_mesh,
        scratch_types=[
            pltpu.VMEM((IRS, IB), jnp.int32),
            pltpu.VMEM((IRS, IB), jnp.int32),
            pltpu.VMEM((CH, H2), jnp.float32),
            pltpu.VMEM((CH, H2), jnp.float32),
            pltpu.SemaphoreType.DMA,
            pltpu.SemaphoreType.DMA,
        ],
        compiler_params=_sc_params,
        name=f"gather_slab{slab}",
    )
    def _gather_sc(a_hbm, b_hbm, src_hbm, dst_hbm, e_out,
                   si_v, di_v, bufa_v, bufb_v, sema, semb):
        cid = lax.axis_index("c")
        sid = lax.axis_index("s")
        wid = sid * NC + cid

        pltpu.sync_copy(src_hbm.at[slab, wid], si_v)
        pltpu.sync_copy(dst_hbm.at[slab, wid], di_v)

        @pl.loop(0, NCHS)
        def _(i):
            ebase = wid * EPWS + i * CH
            copies = []
            for j in range(RB):
                copies.append(pltpu.async_copy(
                    a_hbm.at[si_v.at[i * RB + j]],
                    bufa_v.at[pl.ds(j * IB, IB)], sema))
                copies.append(pltpu.async_copy(
                    b_hbm.at[di_v.at[i * RB + j]],
                    bufb_v.at[pl.ds(j * IB, IB)], semb))
            for c in copies:
                c.wait()
            pltpu.sync_copy(bufa_v, e_out.at[pl.ds(ebase, CH), pl.ds(0, H2)])
            pltpu.sync_copy(bufb_v, e_out.at[pl.ds(ebase, CH), pl.ds(H2, H2)])

    return _gather_sc


_gather_slabs = [_make_gather(s) for s in range(SLABS)]


BN = 1000


def _mlp_body(sums_ref, cnts_ref, x_ref, w0x_ref, w0a_ref, b0_ref,
              g0_ref, be0_ref, w1h_ref, w1a_ref, b1_ref, g1_ref, be1_ref,
              c1a_ref, cb1_ref, c1b_ref, a_ref, bt_ref):
    s = sums_ref[0] + sums_ref[1]
    cnt = cnts_ref[0, :, 0:1] + cnts_ref[1, :, 0:1]
    agg = s / jnp.maximum(cnt, 1.0)

    h = (jnp.dot(x_ref[...], w0x_ref[...], preferred_element_type=jnp.float32)
         + jnp.dot(agg, w0a_ref[...], preferred_element_type=jnp.float32)
         + b0_ref[...])
    h = jnp.maximum(h, 0.0)
    m = jnp.mean(h, axis=-1, keepdims=True)
    v = jnp.mean((h - m) * (h - m), axis=-1, keepdims=True)
    h = (h - m) * lax.rsqrt(v + 1e-5) * g0_ref[...] + be0_ref[...]

    h = (jnp.dot(h, w1h_ref[...], preferred_element_type=jnp.float32)
         + jnp.dot(agg, w1a_ref[...], preferred_element_type=jnp.float32)
         + b1_ref[...])
    h = jnp.maximum(h, 0.0)
    m = jnp.mean(h, axis=-1, keepdims=True)
    v = jnp.mean((h - m) * (h - m), axis=-1, keepdims=True)
    h = (h - m) * lax.rsqrt(v + 1e-5) * g1_ref[...] + be1_ref[...]

    a_ref[...] = (jnp.dot(h, c1a_ref[...], preferred_element_type=jnp.float32)
                  + cb1_ref[...])
    bt_ref[...] = jnp.dot(h, c1b_ref[...], preferred_element_type=jnp.float32)


def _node_mlp(sums, cnts, x, w0x, w0a, b0, g0, be0, w1h, w1a, b1, g1, be1,
              c1a, cb1, c1b):
    full = lambda shape: pl.BlockSpec(shape, lambda i: (0,) * len(shape))
    return pl.pallas_call(
        _mlp_body,
        grid=(N // BN,),
        in_specs=[
            pl.BlockSpec((NC, BN, D), lambda i: (0, i, 0)),
            pl.BlockSpec((NC, BN, 16), lambda i: (0, i, 0)),
            pl.BlockSpec((BN, D), lambda i: (i, 0)),
            full((D, H1)), full((D, H1)), full((1, H1)),
            full((1, H1)), full((1, H1)),
            full((H1, H2)), full((D, H2)), full((1, H2)),
            full((1, H2)), full((1, H2)),
            full((H2, H2)), full((1, H2)), full((H2, H2)),
        ],
        out_specs=[
            pl.BlockSpec((BN, H2), lambda i: (i, 0)),
            pl.BlockSpec((BN, H2), lambda i: (i, 0)),
        ],
        out_shape=[
            jax.ShapeDtypeStruct((N, H2), jnp.float32),
            jax.ShapeDtypeStruct((N, H2), jnp.float32),
        ],
        compiler_params=pltpu.CompilerParams(
            dimension_semantics=("parallel",)),
    )(sums, cnts, x, w0x, w0a, b0, g0, be0, w1h, w1a, b1, g1, be1,
      c1a, cb1, c1b)


BE = 6400


def _cls_body(e_ref, w2_ref, cb2_ref, out_ref):
    e = e_ref[...]
    hid = jnp.maximum(e[:, :H2] + e[:, H2:], 0.0)
    ot = lax.dot_general(w2_ref[...], hid, (((1,), (1,)), ((), ())),
                         preferred_element_type=jnp.float32)
    out_ref[...] = ot + cb2_ref[...]


def _edge_cls(e12, cw2, cb2):
    ne = e12.shape[0]
    return pl.pallas_call(
        _cls_body,
        grid=(ne // BE,),
        in_specs=[
            pl.BlockSpec((BE, 2 * H2), lambda i: (i, 0)),
            pl.BlockSpec((OUT, H2), lambda i: (0, 0)),
            pl.BlockSpec((OUT, 1), lambda i: (0, 0)),
        ],
        out_specs=pl.BlockSpec((OUT, BE), lambda i: (0, i)),
        out_shape=jax.ShapeDtypeStruct((OUT, ne), jnp.float32),
        compiler_params=pltpu.CompilerParams(
            dimension_semantics=("parallel",)),
    )(e12, cw2, cb2)


def kernel(x, edge_index, edge_attr, W0, b0, W1, b1, ln0_g, ln0_b,
           ln1_g, ln1_b, cW1, cb1, cW2, cb2):
    src = edge_index[0].astype(jnp.int32).reshape(SLABS, NW, IRS, IB)
    dst = edge_index[1].astype(jnp.int32).reshape(NW, IR, IB)

    zeros_pad = jnp.zeros((NPAD, DC), jnp.float32)
    sums, cnts = _segsum_sc(edge_attr, dst, zeros_pad)

    a_tab, b_tab = _node_mlp(
        sums, cnts, x,
        W0[:, :D].T, W0[:, D:].T, b0.reshape(1, H1),
        ln0_g.reshape(1, H1), ln0_b.reshape(1, H1),
        W1[:, :H1].T, W1[:, H1:].T, b1.reshape(1, H2),
        ln1_g.reshape(1, H2), ln1_b.reshape(1, H2),
        cW1[:, :H2].T, cb1.reshape(1, H2), cW1[:, H2:].T,
    )

    dst4 = dst.reshape(SLABS, NW, IRS, IB)
    outs = []
    for s in range(SLABS):
        e12 = _gather_slabs[s](a_tab, b_tab, src, dst4)
        outs.append(_edge_cls(e12, cW2, cb2.reshape(OUT, 1)))
    return jnp.concatenate(outs, axis=1).T


# speedup vs baseline: 6.3351x; 1.0113x over previous
import functools

import jax
import jax.numpy as jnp
from jax import lax
from jax.experimental import pallas as pl
from jax.experimental.pallas import tpu as pltpu
from jax.experimental.pallas import tpu_sc as plsc

N = 10000
E = 320000
D = 128
H1 = 128
H2 = 64
OUT = 2

NC = 2
NS = 16
NW = NC * NS
EPW = E // NW
NPAD = 10112
RPS = NPAD // NS
DC = D + 16

IB = 80
IR = EPW // IB
RB = 5
CH = RB * IB
NCH = EPW // CH

_mesh = plsc.VectorSubcoreMesh(core_axis_name="c", subcore_axis_name="s")
_sc_params = pltpu.CompilerParams(use_tc_tiling_on_sc=False)


@functools.partial(
    pl.kernel,
    out_type=(
        jax.ShapeDtypeStruct((NC, NPAD, D), jnp.float32),
        jax.ShapeDtypeStruct((NC, NPAD, 16), jnp.float32),
    ),
    mesh=_mesh,
    scratch_types=[
        pltpu.VMEM((2, IB, DC), jnp.float32),
        pltpu.VMEM((IR, IB), jnp.int32),
        pltpu.VMEM_SHARED((NPAD, DC), jnp.float32),
        pltpu.SemaphoreType.DMA,
        pltpu.SemaphoreType.DMA,
        pltpu.SemaphoreType.DMA,
        pltpu.SemaphoreType.DMA,
    ],
    compiler_params=_sc_params,
)
def _segsum_sc(attr_hbm, dst_hbm, zpad_hbm, sums_out, cnts_out,
               rows_v, idx_v, acc_sh, ssem0, ssem1, asem0, asem1):
    cid = lax.axis_index("c")
    sid = lax.axis_index("s")
    wid = sid * NC + cid
    ssem = (ssem0, ssem1)
    asem = (asem0, asem1)

    pltpu.sync_copy(zpad_hbm.at[pl.ds(sid * RPS, RPS)],
                    acc_sh.at[pl.ds(sid * RPS, RPS)])

    pltpu.sync_copy(dst_hbm.at[wid], idx_v)

    for b in range(2):
        @pl.loop(0, IB)
        def _(r):
            rows_v[b, r, pl.ds(D, 16)] = jnp.full((16,), 1.0, jnp.float32)

    plsc.subcore_barrier()

    def stage(k, g):
        base = wid * EPW + g * CH + k * IB
        return pltpu.async_copy(
            attr_hbm.at[pl.ds(base, IB)],
            rows_v.at[k % 2, slice(None), pl.ds(0, D)], ssem[k % 2])

    def add(k, g):
        return pltpu.async_copy(
            rows_v.at[k % 2], acc_sh.at[idx_v.at[g * RB + k]],
            asem[k % 2], add=True)

    @pl.loop(0, NCH)
    def _(g):
        st = {0: stage(0, g), 1: stage(1, g)}
        ad = {}
        for k in range(RB):
            if k >= 2:
                ad[k - 2].wait()
                st[k] = stage(k, g)
            st[k].wait()
            ad[k] = add(k, g)
        ad[RB - 2].wait()
        ad[RB - 1].wait()

    plsc.subcore_barrier()

    pltpu.sync_copy(acc_sh.at[pl.ds(sid * RPS, RPS), pl.ds(0, D)],
                    sums_out.at[cid, pl.ds(sid * RPS, RPS)])
    pltpu.sync_copy(acc_sh.at[pl.ds(sid * RPS, RPS), pl.ds(D, 16)],
                    cnts_out.at[cid, pl.ds(sid * RPS, RPS)])


SLABS = 5
SLAB_E = E // SLABS
EPWS = SLAB_E // NW
IRS = EPWS // IB
NCHS = EPWS // CH


def _make_gather(slab):
    @functools.partial(
        pl.kernel,
        out_type=jax.ShapeDtypeStruct((SLAB_E, 2 * H2), jnp.float32),
        mesh=_mesh,
        scratch_types=[
            pltpu.VMEM((IRS, IB), jnp.int32),
            pltpu.VMEM((IRS, IB), jnp.int32),
            pltpu.VMEM((2, CH, H2), jnp.float32),
            pltpu.VMEM((2, CH, H2), jnp.float32),
            pltpu.SemaphoreType.DMA,
            pltpu.SemaphoreType.DMA,
            pltpu.SemaphoreType.DMA,
            pltpu.SemaphoreType.DMA,
            pltpu.SemaphoreType.DMA,
            pltpu.SemaphoreType.DMA,
            pltpu.SemaphoreType.DMA,
            pltpu.SemaphoreType.DMA,
        ],
        compiler_params=_sc_params,
        name=f"gather_slab{slab}",
    )
    def _gather_sc(a_hbm, b_hbm, src_hbm, dst_hbm, e_out,
                   si_v, di_v, bufa_v, bufb_v,
                   sga0, sga1, sgb0, sgb1, soa0, soa1, sob0, sob1):
        cid = lax.axis_index("c")
        sid = lax.axis_index("s")
        wid = sid * NC + cid
        sga = (sga0, sga1)
        sgb = (sgb0, sgb1)
        soa = (soa0, soa1)
        sob = (sob0, sob1)

        pltpu.sync_copy(src_hbm.at[slab, wid], si_v)
        pltpu.sync_copy(dst_hbm.at[slab, wid], di_v)

        def fire(c):
            b = c % 2
            lst = []
            for j in range(RB):
                r = c * RB + j
                lst.append(pltpu.async_copy(
                    a_hbm.at[si_v.at[r]],
                    bufa_v.at[b, pl.ds(j * IB, IB)], sga[b]))
                lst.append(pltpu.async_copy(
                    b_hbm.at[di_v.at[r]],
                    bufb_v.at[b, pl.ds(j * IB, IB)], sgb[b]))
            return lst

        def flush(c):
            b = c % 2
            ebase = wid * EPWS + c * CH
            return (
                pltpu.async_copy(
                    bufa_v.at[b],
                    e_out.at[pl.ds(ebase, CH), pl.ds(0, H2)], soa[b]),
                pltpu.async_copy(
                    bufb_v.at[b],
                    e_out.at[pl.ds(ebase, CH), pl.ds(H2, H2)], sob[b]),
            )

        gath = {0: fire(0), 1: fire(1)}
        outs = {}
        for c in range(NCHS):
            for d in gath[c]:
                d.wait()
            outs[c] = flush(c)
            if c + 2 < NCHS:
                outs[c][0].wait()
                outs[c][1].wait()
                gath[c + 2] = fire(c + 2)
        for c in (NCHS - 2, NCHS - 1):
            outs[c][0].wait()
            outs[c][1].wait()

    return _gather_sc


_gather_slabs = [_make_gather(s) for s in range(SLABS)]


BN = 1000


def _mlp_body(sums_ref, cnts_ref, x_ref, w0x_ref, w0a_ref, b0_ref,
              g0_ref, be0_ref, w1h_ref, w1a_ref, b1_ref, g1_ref, be1_ref,
              c1a_ref, cb1_ref, c1b_ref, a_ref, bt_ref):
    s = sums_ref[0] + sums_ref[1]
    cnt = cnts_ref[0, :, 0:1] + cnts_ref[1, :, 0:1]
    agg = s / jnp.maximum(cnt, 1.0)

    h = (jnp.dot(x_ref[...], w0x_ref[...], preferred_element_type=jnp.float32)
         + jnp.dot(agg, w0a_ref[...], preferred_element_type=jnp.float32)
         + b0_ref[...])
    h = jnp.maximum(h, 0.0)
    m = jnp.mean(h, axis=-1, keepdims=True)
    v = jnp.mean((h - m) * (h - m), axis=-1, keepdims=True)
    h = (h - m) * lax.rsqrt(v + 1e-5) * g0_ref[...] + be0_ref[...]

    h = (jnp.dot(h, w1h_ref[...], preferred_element_type=jnp.float32)
         + jnp.dot(agg, w1a_ref[...], preferred_element_type=jnp.float32)
         + b1_ref[...])
    h = jnp.maximum(h, 0.0)
    m = jnp.mean(h, axis=-1, keepdims=True)
    v = jnp.mean((h - m) * (h - m), axis=-1, keepdims=True)
    h = (h - m) * lax.rsqrt(v + 1e-5) * g1_ref[...] + be1_ref[...]

    a_ref[...] = (jnp.dot(h, c1a_ref[...], preferred_element_type=jnp.float32)
                  + cb1_ref[...])
    bt_ref[...] = jnp.dot(h, c1b_ref[...], preferred_element_type=jnp.float32)


def _node_mlp(sums, cnts, x, w0x, w0a, b0, g0, be0, w1h, w1a, b1, g1, be1,
              c1a, cb1, c1b):
    full = lambda shape: pl.BlockSpec(shape, lambda i: (0,) * len(shape))
    return pl.pallas_call(
        _mlp_body,
        grid=(N // BN,),
        in_specs=[
            pl.BlockSpec((NC, BN, D), lambda i: (0, i, 0)),
            pl.BlockSpec((NC, BN, 16), lambda i: (0, i, 0)),
            pl.BlockSpec((BN, D), lambda i: (i, 0)),
            full((D, H1)), full((D, H1)), full((1, H1)),
            full((1, H1)), full((1, H1)),
            full((H1, H2)), full((D, H2)), full((1, H2)),
            full((1, H2)), full((1, H2)),
            full((H2, H2)), full((1, H2)), full((H2, H2)),
        ],
        out_specs=[
            pl.BlockSpec((BN, H2), lambda i: (i, 0)),
            pl.BlockSpec((BN, H2), lambda i: (i, 0)),
        ],
        out_shape=[
            jax.ShapeDtypeStruct((N, H2), jnp.float32),
            jax.ShapeDtypeStruct((N, H2), jnp.float32),
        ],
        compiler_params=pltpu.CompilerParams(
            dimension_semantics=("parallel",)),
    )(sums, cnts, x, w0x, w0a, b0, g0, be0, w1h, w1a, b1, g1, be1,
      c1a, cb1, c1b)


BE = 6400


def _cls_body(e_ref, w2_ref, cb2_ref, out_ref):
    e = e_ref[...]
    hid = jnp.maximum(e[:, :H2] + e[:, H2:], 0.0)
    ot = lax.dot_general(w2_ref[...], hid, (((1,), (1,)), ((), ())),
                         preferred_element_type=jnp.float32)
    out_ref[...] = ot + cb2_ref[...]


def _edge_cls(e12, cw2, cb2):
    ne = e12.shape[0]
    return pl.pallas_call(
        _cls_body,
        grid=(ne // BE,),
        in_specs=[
            pl.BlockSpec((BE, 2 * H2), lambda i: (i, 0)),
            pl.BlockSpec((OUT, H2), lambda i: (0, 0)),
            pl.BlockSpec((OUT, 1), lambda i: (0, 0)),
        ],
        out_specs=pl.BlockSpec((OUT, BE), lambda i: (0, i)),
        out_shape=jax.ShapeDtypeStruct((OUT, ne), jnp.float32),
        compiler_params=pltpu.CompilerParams(
            dimension_semantics=("parallel",)),
    )(e12, cw2, cb2)


def kernel(x, edge_index, edge_attr, W0, b0, W1, b1, ln0_g, ln0_b,
           ln1_g, ln1_b, cW1, cb1, cW2, cb2):
    src = edge_index[0].astype(jnp.int32).reshape(SLABS, NW, IRS, IB)
    dst = edge_index[1].astype(jnp.int32).reshape(NW, IR, IB)

    zeros_pad = jnp.zeros((NPAD, DC), jnp.float32)
    sums, cnts = _segsum_sc(edge_attr, dst, zeros_pad)

    a_tab, b_tab = _node_mlp(
        sums, cnts, x,
        W0[:, :D].T, W0[:, D:].T, b0.reshape(1, H1),
        ln0_g.reshape(1, H1), ln0_b.reshape(1, H1),
        W1[:, :H1].T, W1[:, H1:].T, b1.reshape(1, H2),
        ln1_g.reshape(1, H2), ln1_b.reshape(1, H2),
        cW1[:, :H2].T, cb1.reshape(1, H2), cW1[:, H2:].T,
    )

    dst4 = dst.reshape(SLABS, NW, IRS, IB)
    outs = []
    for s in range(SLABS):
        e12 = _gather_slabs[s](a_tab, b_tab, src, dst4)
        outs.append(_edge_cls(e12, cW2, cb2.reshape(OUT, 1)))
    return jnp.concatenate(outs, axis=1).T


# speedup vs baseline: 6.3682x; 1.0052x over previous
import functools

import jax
import jax.numpy as jnp
from jax import lax
from jax.experimental import pallas as pl
from jax.experimental.pallas import tpu as pltpu
from jax.experimental.pallas import tpu_sc as plsc

N = 10000
E = 320000
D = 128
H1 = 128
H2 = 64
OUT = 2

NC = 2
NS = 16
NW = NC * NS
EPW = E // NW
NPAD = 10112
RPS = NPAD // NS
DC = D + 16

IB = 80
IR = EPW // IB
RB = 5
CH = RB * IB
NCH = EPW // CH

_mesh = plsc.VectorSubcoreMesh(core_axis_name="c", subcore_axis_name="s")
_sc_params = pltpu.CompilerParams(use_tc_tiling_on_sc=False)


@functools.partial(
    pl.kernel,
    out_type=(
        jax.ShapeDtypeStruct((NC, NPAD, D), jnp.float32),
        jax.ShapeDtypeStruct((NC, NPAD, 16), jnp.float32),
    ),
    mesh=_mesh,
    scratch_types=[
        pltpu.VMEM((2, IB, DC), jnp.float32),
        pltpu.VMEM((IR, IB), jnp.int32),
        pltpu.VMEM_SHARED((NPAD, DC), jnp.float32),
        pltpu.SemaphoreType.DMA,
        pltpu.SemaphoreType.DMA,
        pltpu.SemaphoreType.DMA,
        pltpu.SemaphoreType.DMA,
    ],
    compiler_params=_sc_params,
)
def _segsum_sc(attr_hbm, dst_hbm, zpad_hbm, sums_out, cnts_out,
               rows_v, idx_v, acc_sh, ssem0, ssem1, asem0, asem1):
    cid = lax.axis_index("c")
    sid = lax.axis_index("s")
    wid = sid * NC + cid
    ssem = (ssem0, ssem1)
    asem = (asem0, asem1)

    pltpu.sync_copy(zpad_hbm.at[pl.ds(sid * RPS, RPS)],
                    acc_sh.at[pl.ds(sid * RPS, RPS)])

    pltpu.sync_copy(dst_hbm.at[wid], idx_v)

    for b in range(2):
        @pl.loop(0, IB)
        def _(r):
            rows_v[b, r, pl.ds(D, 16)] = jnp.full((16,), 1.0, jnp.float32)

    plsc.subcore_barrier()

    def stage(k, g):
        base = wid * EPW + g * CH + k * IB
        return pltpu.async_copy(
            attr_hbm.at[pl.ds(base, IB)],
            rows_v.at[k % 2, slice(None), pl.ds(0, D)], ssem[k % 2])

    def add(k, g):
        return pltpu.async_copy(
            rows_v.at[k % 2], acc_sh.at[idx_v.at[g * RB + k]],
            asem[k % 2], add=True)

    @pl.loop(0, NCH)
    def _(g):
        st = {0: stage(0, g), 1: stage(1, g)}
        ad = {}
        for k in range(RB):
            if k >= 2:
                ad[k - 2].wait()
                st[k] = stage(k, g)
            st[k].wait()
            ad[k] = add(k, g)
        ad[RB - 2].wait()
        ad[RB - 1].wait()

    plsc.subcore_barrier()

    pltpu.sync_copy(acc_sh.at[pl.ds(sid * RPS, RPS), pl.ds(0, D)],
                    sums_out.at[cid, pl.ds(sid * RPS, RPS)])
    pltpu.sync_copy(acc_sh.at[pl.ds(sid * RPS, RPS), pl.ds(D, 16)],
                    cnts_out.at[cid, pl.ds(sid * RPS, RPS)])


SLABS = 5
SLAB_E = E // SLABS
EPWS = SLAB_E // NW
IRS = EPWS // IB
NCHS = EPWS // CH


def _make_gather(slab):
    @functools.partial(
        pl.kernel,
        out_type=jax.ShapeDtypeStruct((SLAB_E, 2 * H2), jnp.float32),
        mesh=_mesh,
        scratch_types=[
            pltpu.VMEM((IRS, IB), jnp.int32),
            pltpu.VMEM((IRS, IB), jnp.int32),
            pltpu.VMEM((2, CH, H2), jnp.float32),
            pltpu.VMEM((2, CH, H2), jnp.float32),
            pltpu.SemaphoreType.DMA,
            pltpu.SemaphoreType.DMA,
            pltpu.SemaphoreType.DMA,
            pltpu.SemaphoreType.DMA,
            pltpu.SemaphoreType.DMA,
            pltpu.SemaphoreType.DMA,
            pltpu.SemaphoreType.DMA,
            pltpu.SemaphoreType.DMA,
        ],
        compiler_params=_sc_params,
        name=f"gather_slab{slab}",
    )
    def _gather_sc(a_hbm, b_hbm, src_hbm, dst_hbm, e_out,
                   si_v, di_v, bufa_v, bufb_v,
                   sga0, sga1, sgb0, sgb1, soa0, soa1, sob0, sob1):
        cid = lax.axis_index("c")
        sid = lax.axis_index("s")
        wid = sid * NC + cid
        sga = (sga0, sga1)
        sgb = (sgb0, sgb1)
        soa = (soa0, soa1)
        sob = (sob0, sob1)

        pltpu.sync_copy(src_hbm.at[slab, wid], si_v)
        pltpu.sync_copy(dst_hbm.at[slab, wid], di_v)

        def fire(c):
            b = c % 2
            lst = []
            for j in range(RB):
                r = c * RB + j
                lst.append(pltpu.async_copy(
                    a_hbm.at[si_v.at[r]],
                    bufa_v.at[b, pl.ds(j * IB, IB)], sga[b]))
                lst.append(pltpu.async_copy(
                    b_hbm.at[di_v.at[r]],
                    bufb_v.at[b, pl.ds(j * IB, IB)], sgb[b]))
            return lst

        def flush(c):
            b = c % 2
            ebase = wid * EPWS + c * CH
            return (
                pltpu.async_copy(
                    bufa_v.at[b],
                    e_out.at[pl.ds(ebase, CH), pl.ds(0, H2)], soa[b]),
                pltpu.async_copy(
                    bufb_v.at[b],
                    e_out.at[pl.ds(ebase, CH), pl.ds(H2, H2)], sob[b]),
            )

        gath = {0: fire(0), 1: fire(1)}
        outs = {}
        for c in range(NCHS):
            for d in gath[c]:
                d.wait()
            outs[c] = flush(c)
            if c + 2 < NCHS:
                outs[c][0].wait()
                outs[c][1].wait()
                gath[c + 2] = fire(c + 2)
        for c in (NCHS - 2, NCHS - 1):
            outs[c][0].wait()
            outs[c][1].wait()

    return _gather_sc


_gather_slabs = [_make_gather(s) for s in range(SLABS)]


BN = 1000


def _mlp_body(sums_ref, cnts_ref, x_ref, w0x_ref, w0a_ref, b0_ref,
              g0_ref, be0_ref, w1h_ref, w1a_ref, b1_ref, g1_ref, be1_ref,
              c1a_ref, cb1_ref, c1b_ref, a_ref, bt_ref):
    s = sums_ref[0] + sums_ref[1]
    cnt = cnts_ref[0, :, 0:1] + cnts_ref[1, :, 0:1]
    agg = s / jnp.maximum(cnt, 1.0)

    h = (jnp.dot(x_ref[...], w0x_ref[...], preferred_element_type=jnp.float32)
         + jnp.dot(agg, w0a_ref[...], preferred_element_type=jnp.float32)
         + b0_ref[...])
    h = jnp.maximum(h, 0.0)
    m = jnp.mean(h, axis=-1, keepdims=True)
    v = jnp.mean((h - m) * (h - m), axis=-1, keepdims=True)
    h = (h - m) * lax.rsqrt(v + 1e-5) * g0_ref[...] + be0_ref[...]

    h = (jnp.dot(h, w1h_ref[...], preferred_element_type=jnp.float32)
         + jnp.dot(agg, w1a_ref[...], preferred_element_type=jnp.float32)
         + b1_ref[...])
    h = jnp.maximum(h, 0.0)
    m = jnp.mean(h, axis=-1, keepdims=True)
    v = jnp.mean((h - m) * (h - m), axis=-1, keepdims=True)
    h = (h - m) * lax.rsqrt(v + 1e-5) * g1_ref[...] + be1_ref[...]

    a_ref[...] = (jnp.dot(h, c1a_ref[...], preferred_element_type=jnp.float32)
                  + cb1_ref[...])
    bt_ref[...] = jnp.dot(h, c1b_ref[...], preferred_element_type=jnp.float32)


def _node_mlp(sums, cnts, x, w0x, w0a, b0, g0, be0, w1h, w1a, b1, g1, be1,
              c1a, cb1, c1b):
    full = lambda shape: pl.BlockSpec(shape, lambda i: (0,) * len(shape))
    return pl.pallas_call(
        _mlp_body,
        grid=(N // BN,),
        in_specs=[
            pl.BlockSpec((NC, BN, D), lambda i: (0, i, 0)),
            pl.BlockSpec((NC, BN, 16), lambda i: (0, i, 0)),
            pl.BlockSpec((BN, D), lambda i: (i, 0)),
            full((D, H1)), full((D, H1)), full((1, H1)),
            full((1, H1)), full((1, H1)),
            full((H1, H2)), full((D, H2)), full((1, H2)),
            full((1, H2)), full((1, H2)),
            full((H2, H2)), full((1, H2)), full((H2, H2)),
        ],
        out_specs=[
            pl.BlockSpec((BN, H2), lambda i: (i, 0)),
            pl.BlockSpec((BN, H2), lambda i: (i, 0)),
        ],
        out_shape=[
            jax.ShapeDtypeStruct((N, H2), jnp.float32),
            jax.ShapeDtypeStruct((N, H2), jnp.float32),
        ],
        compiler_params=pltpu.CompilerParams(
            dimension_semantics=("parallel",)),
    )(sums, cnts, x, w0x, w0a, b0, g0, be0, w1h, w1a, b1, g1, be1,
      c1a, cb1, c1b)


BE = 12800


def _cls_body(e_ref, w2_ref, cb2_ref, out_ref):
    e = e_ref[...]
    hid = jnp.maximum(e[:, :H2] + e[:, H2:], 0.0)
    ot = lax.dot_general(w2_ref[...], hid, (((1,), (1,)), ((), ())),
                         preferred_element_type=jnp.float32)
    out_ref[...] = ot + cb2_ref[...]


def _edge_cls(e12, cw2, cb2):
    ne = e12.shape[0]
    return pl.pallas_call(
        _cls_body,
        grid=(ne // BE,),
        in_specs=[
            pl.BlockSpec((BE, 2 * H2), lambda i: (i, 0)),
            pl.BlockSpec((OUT, H2), lambda i: (0, 0)),
            pl.BlockSpec((OUT, 1), lambda i: (0, 0)),
        ],
        out_specs=pl.BlockSpec((OUT, BE), lambda i: (0, i)),
        out_shape=jax.ShapeDtypeStruct((OUT, ne), jnp.float32),
        compiler_params=pltpu.CompilerParams(
            dimension_semantics=("parallel",)),
    )(e12, cw2, cb2)


def kernel(x, edge_index, edge_attr, W0, b0, W1, b1, ln0_g, ln0_b,
           ln1_g, ln1_b, cW1, cb1, cW2, cb2):
    src = edge_index[0].astype(jnp.int32).reshape(SLABS, NW, IRS, IB)
    dst = edge_index[1].astype(jnp.int32).reshape(NW, IR, IB)

    zeros_pad = jnp.zeros((NPAD, DC), jnp.float32)
    sums, cnts = _segsum_sc(edge_attr, dst, zeros_pad)

    a_tab, b_tab = _node_mlp(
        sums, cnts, x,
        W0[:, :D].T, W0[:, D:].T, b0.reshape(1, H1),
        ln0_g.reshape(1, H1), ln0_b.reshape(1, H1),
        W1[:, :H1].T, W1[:, H1:].T, b1.reshape(1, H2),
        ln1_g.reshape(1, H2), ln1_b.reshape(1, H2),
        cW1[:, :H2].T, cb1.reshape(1, H2), cW1[:, H2:].T,
    )

    dst4 = dst.reshape(SLABS, NW, IRS, IB)
    outs = []
    for s in range(SLABS):
        e12 = _gather_slabs[s](a_tab, b_tab, src, dst4)
        outs.append(_edge_cls(e12, cW2, cb2.reshape(OUT, 1)))
    return jnp.concatenate(outs, axis=1).T


# speedup vs baseline: 6.7567x; 1.0610x over previous
import functools

import jax
import jax.numpy as jnp
from jax import lax
from jax.experimental import pallas as pl
from jax.experimental.pallas import tpu as pltpu
from jax.experimental.pallas import tpu_sc as plsc

N = 10000
E = 320000
D = 128
H1 = 128
H2 = 64
OUT = 2

NC = 2
NS = 16
NW = NC * NS
EPW = E // NW
NPAD = 10112
RPS = NPAD // NS
DC = D + 16

IB = 80
IR = EPW // IB
RB = 5
CH = RB * IB
NCH = EPW // CH

_mesh = plsc.VectorSubcoreMesh(core_axis_name="c", subcore_axis_name="s")
_sc_params = pltpu.CompilerParams(use_tc_tiling_on_sc=False)


@functools.partial(
    pl.kernel,
    out_type=(
        jax.ShapeDtypeStruct((NC, NPAD, D), jnp.float32),
        jax.ShapeDtypeStruct((NC, NPAD, 16), jnp.float32),
    ),
    mesh=_mesh,
    scratch_types=[
        pltpu.VMEM((3, IB, DC), jnp.float32),
        pltpu.VMEM((2, 24, IB), jnp.int32),
        pltpu.VMEM_SHARED((NPAD, DC), jnp.float32),
        pltpu.SemaphoreType.DMA,
        pltpu.SemaphoreType.DMA,
        pltpu.SemaphoreType.DMA,
        pltpu.SemaphoreType.DMA,
        pltpu.SemaphoreType.DMA,
        pltpu.SemaphoreType.DMA,
    ],
    compiler_params=_sc_params,
)
def _segsum_sc(attr_hbm, dst_hbm, zpad_hbm, sums_out, cnts_out,
               rows_v, idx_v, acc_sh, ss0, ss1, ss2, as0, as1, as2):
    cid = lax.axis_index("c")
    sid = lax.axis_index("s")
    wid = sid * NC + cid
    ssem = (ss0, ss1, ss2)
    asem = (as0, as1, as2)
    SEG = 24

    pltpu.sync_copy(zpad_hbm.at[pl.ds(sid * RPS, RPS)],
                    acc_sh.at[pl.ds(sid * RPS, RPS)])

    for b in range(3):
        @pl.loop(0, IB)
        def _(r):
            rows_v[b, r, pl.ds(D, 16)] = jnp.full((16,), 1.0, jnp.float32)

    plsc.subcore_barrier()

    def stage_row(r):
        base = wid * EPW + r * IB
        return pltpu.async_copy(
            attr_hbm.at[pl.ds(base, IB)],
            rows_v.at[r % 3, slice(None), pl.ds(0, D)], ssem[r % 3])

    def add_row(r):
        return pltpu.async_copy(
            rows_v.at[r % 3],
            acc_sh.at[idx_v.at[(r // SEG) % 2, r % SEG]],
            asem[r % 3], add=True)

    sts = {}
    ads = {}
    for r in range(IR):
        s, off = divmod(r, SEG)
        if off == 0:
            n = min(SEG, IR - r)
            pltpu.sync_copy(dst_hbm.at[wid, pl.ds(r, n)],
                            idx_v.at[s % 2, pl.ds(0, n)])
        if r >= 3:
            ads[r - 3].wait()
        sts[r] = stage_row(r)
        if r >= 2:
            sts[r - 2].wait()
            ads[r - 2] = add_row(r - 2)
    for q in (IR - 2, IR - 1):
        sts[q].wait()
        ads[q] = add_row(q)
    for q in (IR - 3, IR - 2, IR - 1):
        ads[q].wait()

    plsc.subcore_barrier()

    pltpu.sync_copy(acc_sh.at[pl.ds(sid * RPS, RPS), pl.ds(0, D)],
                    sums_out.at[cid, pl.ds(sid * RPS, RPS)])
    pltpu.sync_copy(acc_sh.at[pl.ds(sid * RPS, RPS), pl.ds(D, 16)],
                    cnts_out.at[cid, pl.ds(sid * RPS, RPS)])


SLABS = 5
SLAB_E = E // SLABS
EPWS = SLAB_E // NW
IRS = EPWS // IB
NCHS = EPWS // CH


def _make_gather(slab):
    @functools.partial(
        pl.kernel,
        out_type=jax.ShapeDtypeStruct((SLAB_E, 2 * H2), jnp.float32),
        mesh=_mesh,
        scratch_types=[
            pltpu.VMEM((IRS, IB), jnp.int32),
            pltpu.VMEM((IRS, IB), jnp.int32),
            pltpu.VMEM((2, CH, H2), jnp.float32),
            pltpu.VMEM((2, CH, H2), jnp.float32),
            pltpu.SemaphoreType.DMA,
            pltpu.SemaphoreType.DMA,
            pltpu.SemaphoreType.DMA,
            pltpu.SemaphoreType.DMA,
            pltpu.SemaphoreType.DMA,
            pltpu.SemaphoreType.DMA,
            pltpu.SemaphoreType.DMA,
            pltpu.SemaphoreType.DMA,
        ],
        compiler_params=_sc_params,
        name=f"gather_slab{slab}",
    )
    def _gather_sc(a_hbm, b_hbm, src_hbm, dst_hbm, e_out,
                   si_v, di_v, bufa_v, bufb_v,
                   sga0, sga1, sgb0, sgb1, soa0, soa1, sob0, sob1):
        cid = lax.axis_index("c")
        sid = lax.axis_index("s")
        wid = sid * NC + cid
        sga = (sga0, sga1)
        sgb = (sgb0, sgb1)
        soa = (soa0, soa1)
        sob = (sob0, sob1)

        pltpu.sync_copy(src_hbm.at[slab, wid], si_v)
        pltpu.sync_copy(dst_hbm.at[slab, wid], di_v)

        def fire(c):
            b = c % 2
            lst = []
            for j in range(RB):
                r = c * RB + j
                lst.append(pltpu.async_copy(
                    a_hbm.at[si_v.at[r]],
                    bufa_v.at[b, pl.ds(j * IB, IB)], sga[b]))
                lst.append(pltpu.async_copy(
                    b_hbm.at[di_v.at[r]],
                    bufb_v.at[b, pl.ds(j * IB, IB)], sgb[b]))
            return lst

        def flush(c):
            b = c % 2
            ebase = wid * EPWS + c * CH
            return (
                pltpu.async_copy(
                    bufa_v.at[b],
                    e_out.at[pl.ds(ebase, CH), pl.ds(0, H2)], soa[b]),
                pltpu.async_copy(
                    bufb_v.at[b],
                    e_out.at[pl.ds(ebase, CH), pl.ds(H2, H2)], sob[b]),
            )

        gath = {0: fire(0), 1: fire(1)}
        outs = {}
        for c in range(NCHS):
            for d in gath[c]:
                d.wait()
            outs[c] = flush(c)
            if c + 2 < NCHS:
                outs[c][0].wait()
                outs[c][1].wait()
                gath[c + 2] = fire(c + 2)
        for c in (NCHS - 2, NCHS - 1):
            outs[c][0].wait()
            outs[c][1].wait()

    return _gather_sc


_gather_slabs = [_make_gather(s) for s in range(SLABS)]


BN = 1000


def _mlp_body(sums_ref, cnts_ref, x_ref, w0x_ref, w0a_ref, b0_ref,
              g0_ref, be0_ref, w1h_ref, w1a_ref, b1_ref, g1_ref, be1_ref,
              c1a_ref, cb1_ref, c1b_ref, a_ref, bt_ref):
    s = sums_ref[0] + sums_ref[1]
    cnt = cnts_ref[0, :, 0:1] + cnts_ref[1, :, 0:1]
    agg = s / jnp.maximum(cnt, 1.0)

    h = (jnp.dot(x_ref[...], w0x_ref[...], preferred_element_type=jnp.float32)
         + jnp.dot(agg, w0a_ref[...], preferred_element_type=jnp.float32)
         + b0_ref[...])
    h = jnp.maximum(h, 0.0)
    m = jnp.mean(h, axis=-1, keepdims=True)
    v = jnp.mean((h - m) * (h - m), axis=-1, keepdims=True)
    h = (h - m) * lax.rsqrt(v + 1e-5) * g0_ref[...] + be0_ref[...]

    h = (jnp.dot(h, w1h_ref[...], preferred_element_type=jnp.float32)
         + jnp.dot(agg, w1a_ref[...], preferred_element_type=jnp.float32)
         + b1_ref[...])
    h = jnp.maximum(h, 0.0)
    m = jnp.mean(h, axis=-1, keepdims=True)
    v = jnp.mean((h - m) * (h - m), axis=-1, keepdims=True)
    h = (h - m) * lax.rsqrt(v + 1e-5) * g1_ref[...] + be1_ref[...]

    a_ref[...] = (jnp.dot(h, c1a_ref[...], preferred_element_type=jnp.float32)
                  + cb1_ref[...])
    bt_ref[...] = jnp.dot(h, c1b_ref[...], preferred_element_type=jnp.float32)


def _node_mlp(sums, cnts, x, w0x, w0a, b0, g0, be0, w1h, w1a, b1, g1, be1,
              c1a, cb1, c1b):
    full = lambda shape: pl.BlockSpec(shape, lambda i: (0,) * len(shape))
    return pl.pallas_call(
        _mlp_body,
        grid=(N // BN,),
        in_specs=[
            pl.BlockSpec((NC, BN, D), lambda i: (0, i, 0)),
            pl.BlockSpec((NC, BN, 16), lambda i: (0, i, 0)),
            pl.BlockSpec((BN, D), lambda i: (i, 0)),
            full((D, H1)), full((D, H1)), full((1, H1)),
            full((1, H1)), full((1, H1)),
            full((H1, H2)), full((D, H2)), full((1, H2)),
            full((1, H2)), full((1, H2)),
            full((H2, H2)), full((1, H2)), full((H2, H2)),
        ],
        out_specs=[
            pl.BlockSpec((BN, H2), lambda i: (i, 0)),
            pl.BlockSpec((BN, H2), lambda i: (i, 0)),
        ],
        out_shape=[
            jax.ShapeDtypeStruct((N, H2), jnp.float32),
            jax.ShapeDtypeStruct((N, H2), jnp.float32),
        ],
        compiler_params=pltpu.CompilerParams(
            dimension_semantics=("parallel",)),
    )(sums, cnts, x, w0x, w0a, b0, g0, be0, w1h, w1a, b1, g1, be1,
      c1a, cb1, c1b)


BE = 12800


def _cls_body(e_ref, w2_ref, cb2_ref, out_ref):
    e = e_ref[...]
    hid = jnp.maximum(e[:, :H2] + e[:, H2:], 0.0)
    ot = lax.dot_general(w2_ref[...], hid, (((1,), (1,)), ((), ())),
                         preferred_element_type=jnp.float32)
    out_ref[...] = ot + cb2_ref[...]


def _edge_cls(e12, cw2, cb2):
    ne = e12.shape[0]
    return pl.pallas_call(
        _cls_body,
        grid=(ne // BE,),
        in_specs=[
            pl.BlockSpec((BE, 2 * H2), lambda i: (i, 0)),
            pl.BlockSpec((OUT, H2), lambda i: (0, 0)),
            pl.BlockSpec((OUT, 1), lambda i: (0, 0)),
        ],
        out_specs=pl.BlockSpec((OUT, BE), lambda i: (0, i)),
        out_shape=jax.ShapeDtypeStruct((OUT, ne), jnp.float32),
        compiler_params=pltpu.CompilerParams(
            dimension_semantics=("parallel",)),
    )(e12, cw2, cb2)


def kernel(x, edge_index, edge_attr, W0, b0, W1, b1, ln0_g, ln0_b,
           ln1_g, ln1_b, cW1, cb1, cW2, cb2):
    src = edge_index[0].astype(jnp.int32).reshape(SLABS, NW, IRS, IB)
    dst = edge_index[1].astype(jnp.int32).reshape(NW, IR, IB)

    zeros_pad = jnp.zeros((NPAD, DC), jnp.float32)
    sums, cnts = _segsum_sc(edge_attr, dst, zeros_pad)

    a_tab, b_tab = _node_mlp(
        sums, cnts, x,
        W0[:, :D].T, W0[:, D:].T, b0.reshape(1, H1),
        ln0_g.reshape(1, H1), ln0_b.reshape(1, H1),
        W1[:, :H1].T, W1[:, H1:].T, b1.reshape(1, H2),
        ln1_g.reshape(1, H2), ln1_b.reshape(1, H2),
        cW1[:, :H2].T, cb1.reshape(1, H2), cW1[:, H2:].T,
    )

    dst4 = dst.reshape(SLABS, NW, IRS, IB)
    outs = []
    for s in range(SLABS):
        e12 = _gather_slabs[s](a_tab, b_tab, src, dst4)
        outs.append(_edge_cls(e12, cW2, cb2.reshape(OUT, 1)))
    return jnp.concatenate(outs, axis=1).T
